# Initial kernel scaffold; baseline (speedup 1.0000x reference)
import jax
import jax.numpy as jnp
from jax.experimental import pallas as pl


def kernel(params, clique_attr, x, edge_index, edge_attr, a2c_atom, a2c_clique, tree_edge_index, batch, tree_batch):
    raise NotImplementedError("write your pallas kernel here")


# speedup vs baseline: 2.0119x; 2.0119x over previous
import functools

import jax
import jax.numpy as jnp
from jax import lax
from jax.experimental import pallas as pl
from jax.experimental.pallas import tpu as pltpu
from jax.experimental.pallas import tpu_sc as plsc

H = 256
HALF = 128
NC = 2
NS = 16
CH = 128
L = 3
N_GRAPHS = 256


def _round_up(n, m):
    return ((n + m - 1) // m) * m


def _split2(t):
    n = t.shape[0]
    return t.reshape(n, NC, HALF).transpose(1, 0, 2).reshape(NC * n, HALF)


def _pad_idx(gidx, sidx, n_out, eidx=None):
    e0 = gidx.shape[0]
    ep = _round_up(e0, NS * CH)
    pad = ep - e0
    gidx = jnp.concatenate([gidx.astype(jnp.int32), jnp.zeros((pad,), jnp.int32)])
    sidx = jnp.concatenate([sidx.astype(jnp.int32), jnp.full((pad,), n_out, jnp.int32)])
    if eidx is None:
        return gidx, sidx
    eidx = jnp.concatenate([eidx.astype(jnp.int32), jnp.zeros((pad,), jnp.int32)])
    return gidx, sidx, eidx


@functools.partial(jax.jit, static_argnames=("n_in", "n_out", "n_e", "relu"))
def _sc_seg_sum(tab, gidx, sidx, zeros, etab=None, eidx=None, *, n_in, n_out,
                n_e=0, relu=False):
    e_tot = gidx.shape[0]
    per_sub = e_tot // NS
    n_chunks = per_sub // CH
    n_acc = zeros.shape[0]
    rps = n_acc // NS

    scratch = [
        pltpu.VMEM((CH,), jnp.int32),
        pltpu.VMEM((CH,), jnp.int32),
        pltpu.VMEM((CH, HALF), jnp.float32),
        pltpu.VMEM_SHARED((n_acc, HALF), jnp.float32),
        pltpu.SemaphoreType.DMA,
    ]
    if relu:
        scratch += [
            pltpu.VMEM((CH,), jnp.int32),
            pltpu.VMEM((CH, HALF), jnp.float32),
            pltpu.SemaphoreType.DMA,
        ]

    mesh = plsc.VectorSubcoreMesh(core_axis_name="c", subcore_axis_name="s")

    @functools.partial(
        pl.kernel, mesh=mesh,
        out_type=jax.ShapeDtypeStruct((NC, n_acc, HALF), jnp.float32),
        scratch_types=scratch)
    def k(*refs):
        if relu:
            (tab_h, gidx_h, sidx_h, zeros_h, etab_h, eidx_h, out_h,
             gv, sv, rows, acc, sem, ev, erows, esem) = refs
        else:
            (tab_h, gidx_h, sidx_h, zeros_h, out_h,
             gv, sv, rows, acc, sem) = refs
        c = lax.axis_index("c")
        s = lax.axis_index("s")
        base = s * per_sub
        goff = c * n_in
        pltpu.sync_copy(zeros_h.at[pl.ds(s * rps, rps)],
                        acc.at[pl.ds(s * rps, rps)])
        plsc.subcore_barrier()

        def chunk(i, carry):
            off = base + i * CH
            pltpu.sync_copy(gidx_h.at[pl.ds(off, CH)], gv)
            pltpu.sync_copy(sidx_h.at[pl.ds(off, CH)], sv)
            for j in range(CH // 16):
                sl = pl.ds(j * 16, 16)
                gv[sl] = gv[sl] + goff
            cp = pltpu.async_copy(tab_h.at[gv], rows, sem)
            if relu:
                eoff = c * n_e
                pltpu.sync_copy(eidx_h.at[pl.ds(off, CH)], ev)
                for j in range(CH // 16):
                    sl = pl.ds(j * 16, 16)
                    ev[sl] = ev[sl] + eoff
                ce = pltpu.async_copy(etab_h.at[ev], erows, esem)
            cp.wait()
            if relu:
                ce.wait()

                def row(j, cc):
                    for kk in range(HALF // 16):
                        sl2 = pl.ds(kk * 16, 16)
                        rows[j, sl2] = jnp.maximum(rows[j, sl2] + erows[j, sl2],
                                                   0.0)
                    return cc
                lax.fori_loop(0, CH, row, 0)
            pltpu.sync_copy(rows, acc.at[sv], add=True)
            return carry

        lax.fori_loop(0, n_chunks, chunk, 0)
        plsc.subcore_barrier()
        pltpu.sync_copy(acc.at[pl.ds(s * rps, rps)],
                        out_h.at[c].at[pl.ds(s * rps, rps)])

    if relu:
        out = k(tab, gidx, sidx, zeros, etab, eidx)
    else:
        out = k(tab, gidx, sidx, zeros)
    return out.transpose(1, 0, 2).reshape(n_acc, H)


def _seg_sum(tab256, gidx, sidx, n_in, n_out, etab256=None, eidx=None, n_e=0):
    n_acc = _round_up(n_out + 1, NS * 8)
    zeros = jnp.zeros((n_acc, HALF), jnp.float32)
    if etab256 is None:
        out = _sc_seg_sum(_split2(tab256), gidx, sidx, zeros,
                          n_in=n_in, n_out=n_out)
    else:
        out = _sc_seg_sum(_split2(tab256), gidx, sidx, zeros,
                          _split2(etab256), eidx,
                          n_in=n_in, n_out=n_out, n_e=n_e, relu=True)
    return out[:n_out]


def _bond_tables(bond_emb):
    def body(be_ref, out_ref):
        for l in range(L):
            e0 = be_ref[l, 0]
            e1 = be_ref[l, 1]
            e2 = be_ref[l, 2]
            t = (e0[:, None, None, :] + e1[None, :, None, :]
                 + e2[None, None, :, :])
            out_ref[l] = t.reshape(216, H)
    return pl.pallas_call(
        body, out_shape=jax.ShapeDtypeStruct((L, 216, H), jnp.float32))(bond_emb)


def _clique_enc(attr, emb):
    def body(a_ref, e_ref, o_ref):
        a = a_ref[...]
        mx = jnp.max(a, axis=1, keepdims=True)
        ii = lax.broadcasted_iota(jnp.int32, a.shape, 1)
        pri = jnp.where(a == mx, ii, jnp.int32(10 ** 6))
        amin = jnp.min(pri, axis=1, keepdims=True)
        onehot = (pri == amin).astype(jnp.float32)
        o_ref[...] = jnp.dot(onehot, e_ref[...],
                             preferred_element_type=jnp.float32)
    r = attr.shape[0]
    return pl.pallas_call(
        body, out_shape=jax.ShapeDtypeStruct((r, H), jnp.float32))(attr, emb)


def _mlp(xin, agg, eps, w1, b1, g1, be1, w2, b2, bg, bb):
    def body(e_ref, x_ref, a_ref, w1_ref, b1_ref, g1_ref, be1_ref, w2_ref,
             b2_ref, bg_ref, bb_ref, o_ref):
        z = (1.0 + e_ref[0, 0]) * x_ref[...] + a_ref[...]
        z1 = jnp.dot(z, w1_ref[...], preferred_element_type=jnp.float32)
        z1 = z1 + b1_ref[...]
        m = jnp.mean(z1, axis=0, keepdims=True)
        v = jnp.mean(z1 * z1, axis=0, keepdims=True) - m * m
        z1 = g1_ref[...] * (z1 - m) * lax.rsqrt(v + 1e-5) + be1_ref[...]
        z1 = jnp.maximum(z1, 0.0)
        z2 = jnp.dot(z1, w2_ref[...], preferred_element_type=jnp.float32)
        z2 = z2 + b2_ref[...]
        m2 = jnp.mean(z2, axis=0, keepdims=True)
        v2 = jnp.mean(z2 * z2, axis=0, keepdims=True) - m2 * m2
        o_ref[...] = jnp.maximum(
            bg_ref[...] * (z2 - m2) * lax.rsqrt(v2 + 1e-5) + bb_ref[...], 0.0)
    r = xin.shape[0]
    return pl.pallas_call(
        body, out_shape=jax.ShapeDtypeStruct((r, H), jnp.float32))(
            eps.reshape(1, 1), xin, agg, w1, b1.reshape(1, -1),
            g1.reshape(1, -1), be1.reshape(1, -1), w2, b2.reshape(1, -1),
            bg.reshape(1, -1), bb.reshape(1, -1))


def _mean_lin_add(base, ssum, cnt128, w, b):
    def body(base_ref, s_ref, c_ref, w_ref, b_ref, o_ref):
        c = jnp.maximum(c_ref[...], 1.0)
        m = s_ref[...] / jnp.concatenate([c, c], axis=1)
        o_ref[...] = base_ref[...] + jnp.maximum(
            jnp.dot(m, w_ref[...], preferred_element_type=jnp.float32)
            + b_ref[...], 0.0)
    r = base.shape[0]
    return pl.pallas_call(
        body, out_shape=jax.ShapeDtypeStruct((r, H), jnp.float32))(
            base, ssum, cnt128, w, b.reshape(1, -1))


def _final(hs, hcnt, cs, ccnt, aw, ab, cw, cb, lw, lb):
    def body(hs_ref, hc_ref, cs_ref, cc_ref, aw_ref, ab_ref, cw_ref, cb_ref,
             lw_ref, lb_ref, o_ref):
        hc = jnp.maximum(hc_ref[...], 1.0)
        cc = jnp.maximum(cc_ref[...], 1.0)
        xg = jnp.dot(hs_ref[...] / jnp.concatenate([hc, hc], axis=1),
                     aw_ref[...], preferred_element_type=jnp.float32)
        xg = xg + ab_ref[...]
        xc = jnp.dot(cs_ref[...] / jnp.concatenate([cc, cc], axis=1),
                     cw_ref[...], preferred_element_type=jnp.float32)
        xc = xc + cb_ref[...]
        out = jnp.maximum(xg + xc, 0.0)
        o_ref[...] = jnp.dot(out, lw_ref[...],
                             preferred_element_type=jnp.float32) + lb_ref[...]
    g, out_dim = N_GRAPHS, lw.shape[1]
    return pl.pallas_call(
        body, out_shape=jax.ShapeDtypeStruct((g, out_dim), jnp.float32))(
            hs, hcnt, cs, ccnt, aw, ab.reshape(1, -1), cw, cb.reshape(1, -1),
            lw, lb.reshape(1, -1))


_ONES_TAB = None


def _counts(ids, n_out):
    ones = jnp.ones((NC * 1, HALF), jnp.float32)
    g = jnp.zeros_like(ids)
    gp, sp = _pad_idx(g, ids, n_out)
    n_acc = _round_up(n_out + 1, NS * 8)
    zeros = jnp.zeros((n_acc, HALF), jnp.float32)
    out = _sc_seg_sum(ones, gp, sp, zeros, n_in=1, n_out=n_out)
    return out[:n_out, :HALF]


def kernel(params, clique_attr, x, edge_index, edge_attr, a2c_atom,
           a2c_clique, tree_edge_index, batch, tree_batch):
    p = params
    n = x.shape[0]
    ncl = clique_attr.shape[0]
    g = N_GRAPHS

    enc_g = (x.astype(jnp.int32)
             + 100 * jnp.arange(9, dtype=jnp.int32)[None, :]).reshape(-1)
    enc_s = jnp.repeat(jnp.arange(n, dtype=jnp.int32), 9)
    enc_gp, enc_sp = _pad_idx(enc_g, enc_s, n)

    src, dst = edge_index[0], edge_index[1]
    et = (edge_attr[:, 0] * 36 + edge_attr[:, 1] * 6
          + edge_attr[:, 2]).astype(jnp.int32)
    e_gp, e_sp, e_ep = _pad_idx(src, dst, n, et)

    a2c_gp, a2c_sp = _pad_idx(a2c_atom, a2c_clique, ncl)
    c2a_gp, c2a_sp = _pad_idx(a2c_clique, a2c_atom, n)
    ts, td = tree_edge_index[0], tree_edge_index[1]
    t_gp, t_sp = _pad_idx(ts, td, ncl)
    rb_gp, rb_sp = _pad_idx(jnp.arange(n, dtype=jnp.int32), batch, g)
    rc_gp, rc_sp = _pad_idx(jnp.arange(ncl, dtype=jnp.int32), tree_batch, g)

    cnt_cl = _counts(a2c_clique, ncl)
    cnt_at = _counts(a2c_atom, n)
    cnt_b = _counts(batch, g)
    cnt_tb = _counts(tree_batch, g)

    flat_emb = p['atom_emb'].reshape(9 * 100, H)
    h = _seg_sum(flat_emb, enc_gp, enc_sp, 900, n)
    x_cl = _clique_enc(clique_attr, p['clique_emb'])
    bond_tabs = _bond_tables(p['bond_emb'])

    for l in range(L):
        agg = _seg_sum(h, e_gp, e_sp, n, n, bond_tabs[l], e_ep, 216)
        h = _mlp(h, agg, p['eps_atom'][l], p['atom_w1'][l], p['atom_b1'][l],
                 p['atom_g1'][l], p['atom_be1'][l], p['atom_w2'][l],
                 p['atom_b2'][l], p['atom_bn_g'][l], p['atom_bn_b'][l])
        s_m = _seg_sum(h, a2c_gp, a2c_sp, n, ncl)
        x_cl = _mean_lin_add(x_cl, s_m, cnt_cl, p['a2c_w'][l], p['a2c_b'][l])
        aggc = _seg_sum(x_cl, t_gp, t_sp, ncl, ncl)
        x_cl = _mlp(x_cl, aggc, p['eps_clique'][l], p['cl_w1'][l],
                    p['cl_b1'][l], p['cl_g1'][l], p['cl_be1'][l],
                    p['cl_w2'][l], p['cl_b2'][l], p['cl_bn_g'][l],
                    p['cl_bn_b'][l])
        s_mc = _seg_sum(x_cl, c2a_gp, c2a_sp, ncl, n)
        h = _mean_lin_add(h, s_mc, cnt_at, p['c2a_w'][l], p['c2a_b'][l])

    hs = _seg_sum(h, rb_gp, rb_sp, n, g)
    cs = _seg_sum(x_cl, rc_gp, rc_sp, ncl, g)
    return _final(hs, cnt_b, cs, cnt_tb, p['atom_lin_w'], p['atom_lin_b'],
                  p['clique_lin_w'], p['clique_lin_b'], p['lin_w'], p['lin_b'])


# speedup vs baseline: 2.3123x; 1.1493x over previous
import functools

import jax
import jax.numpy as jnp
from jax import lax
from jax.experimental import pallas as pl
from jax.experimental.pallas import tpu as pltpu
from jax.experimental.pallas import tpu_sc as plsc

H = 256
HALF = 128
NC = 2
NS = 16
CH = 128
L = 3
N_GRAPHS = 256


def _round_up(n, m):
    return ((n + m - 1) // m) * m


def _split2(t):
    n = t.shape[0]
    return t.reshape(n, NC, HALF).transpose(1, 0, 2).reshape(NC * n, HALF)


def _pad_idx(gidx, sidx, n_out, eidx=None):
    e0 = gidx.shape[0]
    ep = _round_up(e0, NS * CH * 4)
    pad = ep - e0
    gidx = jnp.concatenate([gidx.astype(jnp.int32), jnp.zeros((pad,), jnp.int32)])
    sidx = jnp.concatenate([sidx.astype(jnp.int32), jnp.full((pad,), n_out, jnp.int32)])
    if eidx is None:
        return gidx, sidx
    eidx = jnp.concatenate([eidx.astype(jnp.int32), jnp.zeros((pad,), jnp.int32)])
    return gidx, sidx, eidx


def _splitn(t, nsplit):
    n = t.shape[0]
    w = H // nsplit
    return t.reshape(n, nsplit, w).transpose(1, 0, 2).reshape(nsplit * n, w)


@functools.partial(jax.jit, static_argnames=("n_in", "n_out", "n_e", "relu",
                                             "nsplit", "ch"))
def _sc_seg_sum(tab, gidx, sidx, zeros, etab=None, eidx=None, *, n_in, n_out,
                n_e=0, relu=False, nsplit=2, ch=CH):
    e_tot = gidx.shape[0]
    per_sub = e_tot // NS
    n_chunks = per_sub // ch
    width = H // nsplit
    passes = nsplit // NC
    n_acc = zeros.shape[0]
    rps = n_acc // NS

    per_buf = ch * width * (2 if relu else 1) + ch * (3 if relu else 2)
    nbuf = 4 if (2_050_000 - n_acc * width) >= 4 * NS * per_buf else 2
    groups = n_chunks // nbuf

    scratch = []
    for _ in range(nbuf):
        scratch += [pltpu.VMEM((ch,), jnp.int32),
                    pltpu.VMEM((ch,), jnp.int32),
                    pltpu.VMEM((ch, width), jnp.float32),
                    pltpu.SemaphoreType.DMA,
                    pltpu.SemaphoreType.DMA,
                    pltpu.SemaphoreType.DMA,
                    pltpu.SemaphoreType.DMA]
        if relu:
            scratch += [pltpu.VMEM((ch,), jnp.int32),
                        pltpu.VMEM((ch, width), jnp.float32),
                        pltpu.SemaphoreType.DMA,
                        pltpu.SemaphoreType.DMA]
    scratch.append(pltpu.VMEM_SHARED((n_acc, width), jnp.float32))

    mesh = plsc.VectorSubcoreMesh(core_axis_name="c", subcore_axis_name="s")

    @functools.partial(
        pl.kernel, mesh=mesh,
        out_type=jax.ShapeDtypeStruct((nsplit, n_acc, width), jnp.float32),
        scratch_types=scratch)
    def k(*refs):
        if relu:
            (tab_h, gidx_h, sidx_h, zeros_h, etab_h, eidx_h, out_h) = refs[:7]
            sc = refs[7:]
            per = 11
        else:
            (tab_h, gidx_h, sidx_h, zeros_h, out_h) = refs[:5]
            sc = refs[5:]
            per = 7
        gv = [sc[i * per + 0] for i in range(nbuf)]
        sv = [sc[i * per + 1] for i in range(nbuf)]
        rows = [sc[i * per + 2] for i in range(nbuf)]
        sem_ig = [sc[i * per + 3] for i in range(nbuf)]
        sem_is = [sc[i * per + 4] for i in range(nbuf)]
        sem_g = [sc[i * per + 5] for i in range(nbuf)]
        sem_s = [sc[i * per + 6] for i in range(nbuf)]
        if relu:
            ev = [sc[i * per + 7] for i in range(nbuf)]
            erows = [sc[i * per + 8] for i in range(nbuf)]
            sem_ie = [sc[i * per + 9] for i in range(nbuf)]
            sem_ge = [sc[i * per + 10] for i in range(nbuf)]
        acc = sc[nbuf * per]
        c = lax.axis_index("c")
        s = lax.axis_index("s")
        base = s * per_sub

        def idx_issue(b, cn):
            off = base + cn * ch
            pltpu.async_copy(gidx_h.at[pl.ds(off, ch)], gv[b], sem_ig[b])
            pltpu.async_copy(sidx_h.at[pl.ds(off, ch)], sv[b], sem_is[b])
            if relu:
                pltpu.async_copy(eidx_h.at[pl.ds(off, ch)], ev[b], sem_ie[b])

        def idx_wait_adjust(b, goff, eoff):
            pltpu.make_async_copy(gidx_h.at[pl.ds(0, ch)], gv[b],
                                  sem_ig[b]).wait()
            pltpu.make_async_copy(sidx_h.at[pl.ds(0, ch)], sv[b],
                                  sem_is[b]).wait()
            for j in range(ch // 16):
                sl = pl.ds(j * 16, 16)
                gv[b][sl] = gv[b][sl] + goff
            if relu:
                pltpu.make_async_copy(eidx_h.at[pl.ds(0, ch)], ev[b],
                                      sem_ie[b]).wait()
                for j in range(ch // 16):
                    sl = pl.ds(j * 16, 16)
                    ev[b][sl] = ev[b][sl] + eoff

        def gather_issue(b):
            pltpu.async_copy(tab_h.at[gv[b]], rows[b], sem_g[b])
            if relu:
                pltpu.async_copy(etab_h.at[ev[b]], erows[b], sem_ge[b])

        def gather_wait(b):
            pltpu.make_async_copy(tab_h.at[pl.ds(0, ch)], rows[b],
                                  sem_g[b]).wait()
            if relu:
                pltpu.make_async_copy(etab_h.at[pl.ds(0, ch)], erows[b],
                                      sem_ge[b]).wait()

        def compute(b):
            if not relu:
                return

            def rbody(r, cc):
                for u in range(4):
                    for kk in range(width // 16):
                        sl = pl.ds(kk * 16, 16)
                        rows[b][4 * r + u, sl] = jnp.maximum(
                            rows[b][4 * r + u, sl] + erows[b][4 * r + u, sl],
                            0.0)
                return cc
            lax.fori_loop(0, ch // 4, rbody, 0)

        def scatter_issue(b):
            pltpu.async_copy(rows[b], acc.at[sv[b]], sem_s[b], add=True)

        def scatter_wait(b):
            pltpu.make_async_copy(rows[b], acc.at[pl.ds(0, ch)],
                                  sem_s[b]).wait()

        def run_pass(q):
            goff = (c * passes + q) * n_in
            eoff = (c * passes + q) * n_e
            pltpu.sync_copy(zeros_h.at[pl.ds(s * rps, rps)],
                            acc.at[pl.ds(s * rps, rps)])
            plsc.subcore_barrier()
            for b in range(nbuf):
                idx_issue(b, b)
                idx_wait_adjust(b, goff, eoff)
                gather_issue(b)

            def body(g, carry):
                for b in range(nbuf):
                    gather_wait(b)
                    compute(b)
                    scatter_issue(b)
                for b in range(nbuf):
                    scatter_wait(b)
                    idx_issue(b, nbuf * (g + 1) + b)
                for b in range(nbuf):
                    idx_wait_adjust(b, goff, eoff)
                    gather_issue(b)
                return carry
            lax.fori_loop(0, groups - 1, body, 0)
            for b in range(nbuf):
                gather_wait(b)
                compute(b)
                scatter_issue(b)
            for b in range(nbuf):
                scatter_wait(b)
            plsc.subcore_barrier()
            pltpu.sync_copy(acc.at[pl.ds(s * rps, rps)],
                            out_h.at[c * passes + q].at[pl.ds(s * rps, rps)])

        for q in range(passes):
            run_pass(q)

    if relu:
        out = k(tab, gidx, sidx, zeros, etab, eidx)
    else:
        out = k(tab, gidx, sidx, zeros)
    return out.transpose(1, 0, 2).reshape(n_acc, H)


@functools.partial(jax.jit, static_argnames=("n_out",))
def _sc_counts(sidx, ones, zeros, *, n_out):
    e_tot = sidx.shape[0]
    per_w = e_tot // (NC * NS)
    n_chunks = per_w // CH
    nbuf = 2
    groups = n_chunks // nbuf
    n_acc = zeros.shape[0]
    rps = n_acc // NS

    scratch = [pltpu.VMEM((CH, HALF), jnp.float32)]
    for _ in range(nbuf):
        scratch += [pltpu.VMEM((CH,), jnp.int32),
                    pltpu.SemaphoreType.DMA,
                    pltpu.SemaphoreType.DMA]
    scratch.append(pltpu.VMEM_SHARED((n_acc, HALF), jnp.float32))

    mesh = plsc.VectorSubcoreMesh(core_axis_name="c", subcore_axis_name="s")

    @functools.partial(
        pl.kernel, mesh=mesh,
        out_type=jax.ShapeDtypeStruct((NC, n_acc, HALF), jnp.float32),
        scratch_types=scratch)
    def k(sidx_h, ones_h, zeros_h, out_h, *sc):
        rows = sc[0]
        sv = [sc[1 + 3 * i] for i in range(nbuf)]
        sem_is = [sc[2 + 3 * i] for i in range(nbuf)]
        sem_s = [sc[3 + 3 * i] for i in range(nbuf)]
        acc = sc[1 + 3 * nbuf]
        c = lax.axis_index("c")
        s = lax.axis_index("s")
        base = (c * NS + s) * per_w
        pltpu.sync_copy(ones_h, rows)
        pltpu.sync_copy(zeros_h.at[pl.ds(s * rps, rps)],
                        acc.at[pl.ds(s * rps, rps)])
        plsc.subcore_barrier()

        def idx_issue(b, cn):
            pltpu.async_copy(sidx_h.at[pl.ds(base + cn * CH, CH)], sv[b],
                             sem_is[b])

        def idx_wait(b):
            pltpu.make_async_copy(sidx_h.at[pl.ds(0, CH)], sv[b],
                                  sem_is[b]).wait()

        def scatter_issue(b):
            pltpu.async_copy(rows, acc.at[sv[b]], sem_s[b], add=True)

        def scatter_wait(b):
            pltpu.make_async_copy(rows, acc.at[pl.ds(0, CH)], sem_s[b]).wait()

        for b in range(nbuf):
            idx_issue(b, b)

        def body(g, carry):
            for b in range(nbuf):
                idx_wait(b)
                scatter_issue(b)
            for b in range(nbuf):
                scatter_wait(b)
                idx_issue(b, nbuf * (g + 1) + b)
            return carry
        lax.fori_loop(0, groups - 1, body, 0)
        for b in range(nbuf):
            idx_wait(b)
            scatter_issue(b)
        for b in range(nbuf):
            scatter_wait(b)
        plsc.subcore_barrier()
        pltpu.sync_copy(acc.at[pl.ds(s * rps, rps)],
                        out_h.at[c].at[pl.ds(s * rps, rps)])

    return k(sidx, ones, zeros)


def _seg_sum(tab256, gidx, sidx, n_in, n_out, etab256=None, eidx=None, n_e=0,
             nsplit=2, ch=CH):
    n_acc = _round_up(n_out + 1, NS * 8)
    zeros = jnp.zeros((n_acc, H // nsplit), jnp.float32)
    if etab256 is None:
        out = _sc_seg_sum(_splitn(tab256, nsplit), gidx, sidx, zeros,
                          n_in=n_in, n_out=n_out, nsplit=nsplit, ch=ch)
    else:
        out = _sc_seg_sum(_splitn(tab256, nsplit), gidx, sidx, zeros,
                          _splitn(etab256, nsplit), eidx,
                          n_in=n_in, n_out=n_out, n_e=n_e, relu=True,
                          nsplit=nsplit, ch=ch)
    return out[:n_out]


def _bond_tables(bond_emb):
    def body(be_ref, out_ref):
        for l in range(L):
            e0 = be_ref[l, 0]
            e1 = be_ref[l, 1]
            e2 = be_ref[l, 2]
            t = (e0[:, None, None, :] + e1[None, :, None, :]
                 + e2[None, None, :, :])
            out_ref[l] = t.reshape(216, H)
    return pl.pallas_call(
        body, out_shape=jax.ShapeDtypeStruct((L, 216, H), jnp.float32))(bond_emb)


def _clique_enc(attr, emb):
    def body(a_ref, e_ref, o_ref):
        a = a_ref[...]
        mx = jnp.max(a, axis=1, keepdims=True)
        ii = lax.broadcasted_iota(jnp.int32, a.shape, 1)
        pri = jnp.where(a == mx, ii, jnp.int32(10 ** 6))
        amin = jnp.min(pri, axis=1, keepdims=True)
        onehot = (pri == amin).astype(jnp.float32)
        o_ref[...] = jnp.dot(onehot, e_ref[...],
                             preferred_element_type=jnp.float32)
    r = attr.shape[0]
    return pl.pallas_call(
        body, out_shape=jax.ShapeDtypeStruct((r, H), jnp.float32))(attr, emb)


def _mlp(xin, agg, eps, w1, b1, g1, be1, w2, b2, bg, bb):
    def body(e_ref, x_ref, a_ref, w1_ref, b1_ref, g1_ref, be1_ref, w2_ref,
             b2_ref, bg_ref, bb_ref, o_ref):
        z = (1.0 + e_ref[0, 0]) * x_ref[...] + a_ref[...]
        z1 = jnp.dot(z, w1_ref[...], preferred_element_type=jnp.float32)
        z1 = z1 + b1_ref[...]
        m = jnp.mean(z1, axis=0, keepdims=True)
        v = jnp.mean(z1 * z1, axis=0, keepdims=True) - m * m
        z1 = g1_ref[...] * (z1 - m) * lax.rsqrt(v + 1e-5) + be1_ref[...]
        z1 = jnp.maximum(z1, 0.0)
        z2 = jnp.dot(z1, w2_ref[...], preferred_element_type=jnp.float32)
        z2 = z2 + b2_ref[...]
        m2 = jnp.mean(z2, axis=0, keepdims=True)
        v2 = jnp.mean(z2 * z2, axis=0, keepdims=True) - m2 * m2
        o_ref[...] = jnp.maximum(
            bg_ref[...] * (z2 - m2) * lax.rsqrt(v2 + 1e-5) + bb_ref[...], 0.0)
    r = xin.shape[0]
    return pl.pallas_call(
        body, out_shape=jax.ShapeDtypeStruct((r, H), jnp.float32))(
            eps.reshape(1, 1), xin, agg, w1, b1.reshape(1, -1),
            g1.reshape(1, -1), be1.reshape(1, -1), w2, b2.reshape(1, -1),
            bg.reshape(1, -1), bb.reshape(1, -1))


def _mean_lin_add(base, ssum, cnt2, w, b):
    def body(base_ref, s_ref, ca_ref, cb_ref, w_ref, b_ref, o_ref):
        c = jnp.maximum(ca_ref[...] + cb_ref[...], 1.0)
        m = s_ref[...] / jnp.concatenate([c, c], axis=1)
        o_ref[...] = base_ref[...] + jnp.maximum(
            jnp.dot(m, w_ref[...], preferred_element_type=jnp.float32)
            + b_ref[...], 0.0)
    r = base.shape[0]
    return pl.pallas_call(
        body, out_shape=jax.ShapeDtypeStruct((r, H), jnp.float32))(
            base, ssum, cnt2[0], cnt2[1], w, b.reshape(1, -1))


def _final(hs, hcnt2, cs, ccnt2, aw, ab, cw, cb, lw, lb):
    def body(hs_ref, hca_ref, hcb_ref, cs_ref, cca_ref, ccb_ref, aw_ref,
             ab_ref, cw_ref, cb_ref, lw_ref, lb_ref, o_ref):
        hc = jnp.maximum(hca_ref[...] + hcb_ref[...], 1.0)
        cc = jnp.maximum(cca_ref[...] + ccb_ref[...], 1.0)
        xg = jnp.dot(hs_ref[...] / jnp.concatenate([hc, hc], axis=1),
                     aw_ref[...], preferred_element_type=jnp.float32)
        xg = xg + ab_ref[...]
        xc = jnp.dot(cs_ref[...] / jnp.concatenate([cc, cc], axis=1),
                     cw_ref[...], preferred_element_type=jnp.float32)
        xc = xc + cb_ref[...]
        out = jnp.maximum(xg + xc, 0.0)
        o_ref[...] = jnp.dot(out, lw_ref[...],
                             preferred_element_type=jnp.float32) + lb_ref[...]
    g, out_dim = N_GRAPHS, lw.shape[1]
    return pl.pallas_call(
        body, out_shape=jax.ShapeDtypeStruct((g, out_dim), jnp.float32))(
            hs, hcnt2[0], hcnt2[1], cs, ccnt2[0], ccnt2[1], aw,
            ab.reshape(1, -1), cw, cb.reshape(1, -1), lw, lb.reshape(1, -1))


def kernel(params, clique_attr, x, edge_index, edge_attr, a2c_atom,
           a2c_clique, tree_edge_index, batch, tree_batch):
    p = params
    n = x.shape[0]
    ncl = clique_attr.shape[0]
    g = N_GRAPHS

    enc_g = (x.astype(jnp.int32)
             + 100 * jnp.arange(9, dtype=jnp.int32)[None, :]).reshape(-1)
    enc_s = jnp.repeat(jnp.arange(n, dtype=jnp.int32), 9)
    enc_gp, enc_sp = _pad_idx(enc_g, enc_s, n)

    src, dst = edge_index[0], edge_index[1]
    et = (edge_attr[:, 0] * 36 + edge_attr[:, 1] * 6
          + edge_attr[:, 2]).astype(jnp.int32)
    e_gp, e_sp, e_ep = _pad_idx(src, dst, n, et)

    a2c_gp, a2c_sp = _pad_idx(a2c_atom, a2c_clique, ncl)
    c2a_gp, c2a_sp = _pad_idx(a2c_clique, a2c_atom, n)
    ts, td = tree_edge_index[0], tree_edge_index[1]
    t_gp, t_sp = _pad_idx(ts, td, ncl)
    ro_g = jnp.concatenate([jnp.arange(n, dtype=jnp.int32),
                            n + jnp.arange(ncl, dtype=jnp.int32)])
    ro_s = jnp.concatenate([batch.astype(jnp.int32),
                            g + tree_batch.astype(jnp.int32)])
    ro_gp, ro_sp = _pad_idx(ro_g, ro_s, 2 * g)

    off_at = ncl
    off_b = ncl + n
    off_tb = ncl + n + g
    n_out_c = ncl + n + 2 * g
    cnt_ids = jnp.concatenate([
        a2c_clique.astype(jnp.int32),
        off_at + a2c_atom.astype(jnp.int32),
        off_b + batch.astype(jnp.int32),
        off_tb + tree_batch.astype(jnp.int32)])
    ep = _round_up(cnt_ids.shape[0], NS * CH * 4)
    cnt_ids = jnp.concatenate([
        cnt_ids, jnp.full((ep - cnt_ids.shape[0],), n_out_c, jnp.int32)])
    n_acc_c = _round_up(n_out_c + 1, NS * 8)
    cnts = _sc_counts(cnt_ids, jnp.ones((CH, HALF), jnp.float32),
                      jnp.zeros((n_acc_c, HALF), jnp.float32), n_out=n_out_c)
    cnt_cl = (cnts[0, 0:ncl], cnts[1, 0:ncl])
    cnt_at = (cnts[0, off_at:off_at + n], cnts[1, off_at:off_at + n])
    cnt_b = (cnts[0, off_b:off_b + g], cnts[1, off_b:off_b + g])
    cnt_tb = (cnts[0, off_tb:off_tb + g], cnts[1, off_tb:off_tb + g])

    flat_emb = p['atom_emb'].reshape(9 * 100, H)
    h = _seg_sum(flat_emb, enc_gp, enc_sp, 900, n)
    x_cl = _clique_enc(clique_attr, p['clique_emb'])
    bond_tabs = _bond_tables(p['bond_emb'])

    for l in range(L):
        agg = _seg_sum(h, e_gp, e_sp, n, n, bond_tabs[l], e_ep, 216, ch=64)
        h = _mlp(h, agg, p['eps_atom'][l], p['atom_w1'][l], p['atom_b1'][l],
                 p['atom_g1'][l], p['atom_be1'][l], p['atom_w2'][l],
                 p['atom_b2'][l], p['atom_bn_g'][l], p['atom_bn_b'][l])
        s_m = _seg_sum(h, a2c_gp, a2c_sp, n, ncl)
        x_cl = _mean_lin_add(x_cl, s_m, cnt_cl, p['a2c_w'][l], p['a2c_b'][l])
        aggc = _seg_sum(x_cl, t_gp, t_sp, ncl, ncl)
        x_cl = _mlp(x_cl, aggc, p['eps_clique'][l], p['cl_w1'][l],
                    p['cl_b1'][l], p['cl_g1'][l], p['cl_be1'][l],
                    p['cl_w2'][l], p['cl_b2'][l], p['cl_bn_g'][l],
                    p['cl_bn_b'][l])
        s_mc = _seg_sum(x_cl, c2a_gp, c2a_sp, ncl, n)
        h = _mean_lin_add(h, s_mc, cnt_at, p['c2a_w'][l], p['c2a_b'][l])

    ro = _seg_sum(jnp.concatenate([h, x_cl], axis=0), ro_gp, ro_sp,
                  n + ncl, 2 * g)
    hs = ro[:g]
    cs = ro[g:]
    return _final(hs, cnt_b, cs, cnt_tb, p['atom_lin_w'], p['atom_lin_b'],
                  p['clique_lin_w'], p['clique_lin_b'], p['lin_w'], p['lin_b'])


# speedup vs baseline: 4.6436x; 2.0083x over previous
import functools

import jax
import jax.numpy as jnp
from jax import lax
from jax.experimental import pallas as pl
from jax.experimental.pallas import tpu as pltpu
from jax.experimental.pallas import tpu_sc as plsc

H = 256
HALF = 128
NC = 2
NS = 16
CH = 128
L = 3
N_GRAPHS = 256


def _round_up(n, m):
    return ((n + m - 1) // m) * m


def _split2(t):
    n = t.shape[0]
    return t.reshape(n, NC, HALF).transpose(1, 0, 2).reshape(NC * n, HALF)


def _pad_idx(gidx, sidx, n_out, eidx=None, unit=NS * CH * 2):
    e0 = gidx.shape[0]
    ep = _round_up(e0, unit)
    pad = ep - e0
    gidx = jnp.concatenate([gidx.astype(jnp.int32), jnp.zeros((pad,), jnp.int32)])
    sidx = jnp.concatenate([sidx.astype(jnp.int32), jnp.full((pad,), n_out, jnp.int32)])
    if eidx is None:
        return gidx, sidx
    eidx = jnp.concatenate([eidx.astype(jnp.int32), jnp.zeros((pad,), jnp.int32)])
    return gidx, sidx, eidx


def _splitn(t, nsplit):
    n = t.shape[0]
    w = H // nsplit
    return t.reshape(n, nsplit, w).transpose(1, 0, 2).reshape(nsplit * n, w)


@functools.partial(jax.jit, static_argnames=("n_in", "n_out", "n_e", "relu",
                                             "nsplit", "ch"))
def _sc_seg_sum(tab, gidx, sidx, zeros, etab=None, eidx=None, *, n_in, n_out,
                n_e=0, relu=False, nsplit=2, ch=CH):
    e_tot = gidx.shape[0] // NC
    per_sub = e_tot // NS
    n_chunks = per_sub // ch
    width = H // nsplit
    n_acc = zeros.shape[0]
    rps = n_acc // NS

    per_buf = ch * width * (2 if relu else 1) + ch * (3 if relu else 2)
    nbuf = (4 if (2_050_000 - n_acc * width) >= 4 * NS * per_buf
            and n_chunks % 4 == 0 else 2)
    groups = n_chunks // nbuf

    scratch = []
    for _ in range(nbuf):
        scratch += [pltpu.VMEM((ch,), jnp.int32),
                    pltpu.VMEM((ch,), jnp.int32),
                    pltpu.VMEM((ch, width), jnp.float32),
                    pltpu.SemaphoreType.DMA,
                    pltpu.SemaphoreType.DMA,
                    pltpu.SemaphoreType.DMA,
                    pltpu.SemaphoreType.DMA]
        if relu:
            scratch += [pltpu.VMEM((ch,), jnp.int32),
                        pltpu.VMEM((ch, width), jnp.float32),
                        pltpu.SemaphoreType.DMA,
                        pltpu.SemaphoreType.DMA]
    scratch.append(pltpu.VMEM_SHARED((n_acc, width), jnp.float32))

    mesh = plsc.VectorSubcoreMesh(core_axis_name="c", subcore_axis_name="s")

    @functools.partial(
        pl.kernel, mesh=mesh,
        out_type=jax.ShapeDtypeStruct((NC, n_acc, width), jnp.float32),
        scratch_types=scratch)
    def k(*refs):
        if relu:
            (tab_h, gidx_h, sidx_h, zeros_h, etab_h, eidx_h, out_h) = refs[:7]
            sc = refs[7:]
            per = 11
        else:
            (tab_h, gidx_h, sidx_h, zeros_h, out_h) = refs[:5]
            sc = refs[5:]
            per = 7
        gv = [sc[i * per + 0] for i in range(nbuf)]
        sv = [sc[i * per + 1] for i in range(nbuf)]
        rows = [sc[i * per + 2] for i in range(nbuf)]
        sem_ig = [sc[i * per + 3] for i in range(nbuf)]
        sem_is = [sc[i * per + 4] for i in range(nbuf)]
        sem_g = [sc[i * per + 5] for i in range(nbuf)]
        sem_s = [sc[i * per + 6] for i in range(nbuf)]
        if relu:
            ev = [sc[i * per + 7] for i in range(nbuf)]
            erows = [sc[i * per + 8] for i in range(nbuf)]
            sem_ie = [sc[i * per + 9] for i in range(nbuf)]
            sem_ge = [sc[i * per + 10] for i in range(nbuf)]
        acc = sc[nbuf * per]
        c = lax.axis_index("c")
        s = lax.axis_index("s")
        base = s * per_sub
        cbase = c * e_tot

        def idx_issue(b, cn):
            off = base + cn * ch
            pltpu.async_copy(gidx_h.at[pl.ds(cbase + off, ch)], gv[b],
                             sem_ig[b])
            pltpu.async_copy(sidx_h.at[pl.ds(off, ch)], sv[b], sem_is[b])
            if relu:
                pltpu.async_copy(eidx_h.at[pl.ds(cbase + off, ch)], ev[b],
                                 sem_ie[b])

        def idx_wait(b):
            pltpu.make_async_copy(gidx_h.at[pl.ds(0, ch)], gv[b],
                                  sem_ig[b]).wait()
            pltpu.make_async_copy(sidx_h.at[pl.ds(0, ch)], sv[b],
                                  sem_is[b]).wait()
            if relu:
                pltpu.make_async_copy(eidx_h.at[pl.ds(0, ch)], ev[b],
                                      sem_ie[b]).wait()

        def gather_issue(b):
            pltpu.async_copy(tab_h.at[gv[b]], rows[b], sem_g[b])
            if relu:
                pltpu.async_copy(etab_h.at[ev[b]], erows[b], sem_ge[b])

        def gather_wait(b):
            pltpu.make_async_copy(tab_h.at[pl.ds(0, ch)], rows[b],
                                  sem_g[b]).wait()
            if relu:
                pltpu.make_async_copy(etab_h.at[pl.ds(0, ch)], erows[b],
                                      sem_ge[b]).wait()

        def compute(b):
            if not relu:
                return

            def rbody(r, cc):
                for u in range(4):
                    for kk in range(width // 16):
                        sl = pl.ds(kk * 16, 16)
                        rows[b][4 * r + u, sl] = jnp.maximum(
                            rows[b][4 * r + u, sl] + erows[b][4 * r + u, sl],
                            0.0)
                return cc
            lax.fori_loop(0, ch // 4, rbody, 0)

        def scatter_issue(b):
            pltpu.async_copy(rows[b], acc.at[sv[b]], sem_s[b], add=True)

        def scatter_wait(b):
            pltpu.make_async_copy(rows[b], acc.at[pl.ds(0, ch)],
                                  sem_s[b]).wait()

        pltpu.sync_copy(zeros_h.at[pl.ds(s * rps, rps)],
                        acc.at[pl.ds(s * rps, rps)])
        plsc.subcore_barrier()
        for b in range(nbuf):
            idx_issue(b, b)
            idx_wait(b)
            gather_issue(b)

        def body(g, carry):
            for b in range(nbuf):
                gather_wait(b)
                compute(b)
                scatter_issue(b)
            for b in range(nbuf):
                scatter_wait(b)
                idx_issue(b, nbuf * (g + 1) + b)
            for b in range(nbuf):
                idx_wait(b)
                gather_issue(b)
            return carry
        lax.fori_loop(0, groups - 1, body, 0)
        for b in range(nbuf):
            gather_wait(b)
            compute(b)
            scatter_issue(b)
        for b in range(nbuf):
            scatter_wait(b)
        plsc.subcore_barrier()
        pltpu.sync_copy(acc.at[pl.ds(s * rps, rps)],
                        out_h.at[c].at[pl.ds(s * rps, rps)])

    if relu:
        out = k(tab, gidx, sidx, zeros, etab, eidx)
    else:
        out = k(tab, gidx, sidx, zeros)
    return out.transpose(1, 0, 2).reshape(n_acc, H)


@functools.partial(jax.jit, static_argnames=("n_out",))
def _sc_counts(sidx, ones, zeros, *, n_out):
    e_tot = sidx.shape[0]
    per_w = e_tot // (NC * NS)
    n_chunks = per_w // CH
    nbuf = 2
    groups = n_chunks // nbuf
    n_acc = zeros.shape[0]
    rps = n_acc // NS

    scratch = [pltpu.VMEM((CH, HALF), jnp.float32)]
    for _ in range(nbuf):
        scratch += [pltpu.VMEM((CH,), jnp.int32),
                    pltpu.SemaphoreType.DMA,
                    pltpu.SemaphoreType.DMA]
    scratch.append(pltpu.VMEM_SHARED((n_acc, HALF), jnp.float32))

    mesh = plsc.VectorSubcoreMesh(core_axis_name="c", subcore_axis_name="s")

    @functools.partial(
        pl.kernel, mesh=mesh,
        out_type=jax.ShapeDtypeStruct((NC, n_acc, HALF), jnp.float32),
        scratch_types=scratch)
    def k(sidx_h, ones_h, zeros_h, out_h, *sc):
        rows = sc[0]
        sv = [sc[1 + 3 * i] for i in range(nbuf)]
        sem_is = [sc[2 + 3 * i] for i in range(nbuf)]
        sem_s = [sc[3 + 3 * i] for i in range(nbuf)]
        acc = sc[1 + 3 * nbuf]
        c = lax.axis_index("c")
        s = lax.axis_index("s")
        base = (c * NS + s) * per_w
        pltpu.sync_copy(ones_h, rows)
        pltpu.sync_copy(zeros_h.at[pl.ds(s * rps, rps)],
                        acc.at[pl.ds(s * rps, rps)])
        plsc.subcore_barrier()

        def idx_issue(b, cn):
            pltpu.async_copy(sidx_h.at[pl.ds(base + cn * CH, CH)], sv[b],
                             sem_is[b])

        def idx_wait(b):
            pltpu.make_async_copy(sidx_h.at[pl.ds(0, CH)], sv[b],
                                  sem_is[b]).wait()

        def scatter_issue(b):
            pltpu.async_copy(rows, acc.at[sv[b]], sem_s[b], add=True)

        def scatter_wait(b):
            pltpu.make_async_copy(rows, acc.at[pl.ds(0, CH)], sem_s[b]).wait()

        for b in range(nbuf):
            idx_issue(b, b)

        def body(g, carry):
            for b in range(nbuf):
                idx_wait(b)
                scatter_issue(b)
            for b in range(nbuf):
                scatter_wait(b)
                idx_issue(b, nbuf * (g + 1) + b)
            return carry
        lax.fori_loop(0, groups - 1, body, 0)
        for b in range(nbuf):
            idx_wait(b)
            scatter_issue(b)
        for b in range(nbuf):
            scatter_wait(b)
        plsc.subcore_barrier()
        pltpu.sync_copy(acc.at[pl.ds(s * rps, rps)],
                        out_h.at[c].at[pl.ds(s * rps, rps)])

    return k(sidx, ones, zeros)


def _seg_sum(tab256, gidx, sidx, n_in, n_out, etab256=None, eidx=None, n_e=0,
             nsplit=2, ch=CH):
    n_acc = _round_up(n_out + 1, NS * 8)
    zeros = jnp.zeros((n_acc, H // nsplit), jnp.float32)
    g2 = jnp.concatenate([gidx, gidx + n_in])
    if etab256 is None:
        out = _sc_seg_sum(_splitn(tab256, nsplit), g2, sidx, zeros,
                          n_in=n_in, n_out=n_out, nsplit=nsplit, ch=ch)
    else:
        e2 = jnp.concatenate([eidx, eidx + n_e])
        out = _sc_seg_sum(_splitn(tab256, nsplit), g2, sidx, zeros,
                          _splitn(etab256, nsplit), e2,
                          n_in=n_in, n_out=n_out, n_e=n_e, relu=True,
                          nsplit=nsplit, ch=ch)
    return out[:n_out]


def _bond_tables(bond_emb):
    def body(be_ref, out_ref):
        for l in range(L):
            e0 = be_ref[l, 0]
            e1 = be_ref[l, 1]
            e2 = be_ref[l, 2]
            t = (e0[:, None, None, :] + e1[None, :, None, :]
                 + e2[None, None, :, :])
            out_ref[l] = t.reshape(216, H)
    return pl.pallas_call(
        body, out_shape=jax.ShapeDtypeStruct((L, 216, H), jnp.float32))(bond_emb)


def _clique_enc(attr, emb):
    def body(a_ref, e_ref, o_ref):
        a = a_ref[...]
        mx = jnp.max(a, axis=1, keepdims=True)
        ii = lax.broadcasted_iota(jnp.int32, a.shape, 1)
        pri = jnp.where(a == mx, ii, jnp.int32(10 ** 6))
        amin = jnp.min(pri, axis=1, keepdims=True)
        onehot = (pri == amin).astype(jnp.float32)
        o_ref[...] = jnp.dot(onehot, e_ref[...],
                             preferred_element_type=jnp.float32)
    r = attr.shape[0]
    return pl.pallas_call(
        body, out_shape=jax.ShapeDtypeStruct((r, H), jnp.float32))(attr, emb)


def _mlp(xin, agg, eps, w1, b1, g1, be1, w2, b2, bg, bb):
    def body(e_ref, x_ref, a_ref, w1_ref, b1_ref, g1_ref, be1_ref, w2_ref,
             b2_ref, bg_ref, bb_ref, o_ref):
        z = (1.0 + e_ref[0, 0]) * x_ref[...] + a_ref[...]
        z1 = jnp.dot(z, w1_ref[...], preferred_element_type=jnp.float32)
        z1 = z1 + b1_ref[...]
        m = jnp.mean(z1, axis=0, keepdims=True)
        v = jnp.mean(z1 * z1, axis=0, keepdims=True) - m * m
        z1 = g1_ref[...] * (z1 - m) * lax.rsqrt(v + 1e-5) + be1_ref[...]
        z1 = jnp.maximum(z1, 0.0)
        z2 = jnp.dot(z1, w2_ref[...], preferred_element_type=jnp.float32)
        z2 = z2 + b2_ref[...]
        m2 = jnp.mean(z2, axis=0, keepdims=True)
        v2 = jnp.mean(z2 * z2, axis=0, keepdims=True) - m2 * m2
        o_ref[...] = jnp.maximum(
            bg_ref[...] * (z2 - m2) * lax.rsqrt(v2 + 1e-5) + bb_ref[...], 0.0)
    r = xin.shape[0]
    return pl.pallas_call(
        body, out_shape=jax.ShapeDtypeStruct((r, H), jnp.float32))(
            eps.reshape(1, 1), xin, agg, w1, b1.reshape(1, -1),
            g1.reshape(1, -1), be1.reshape(1, -1), w2, b2.reshape(1, -1),
            bg.reshape(1, -1), bb.reshape(1, -1))


def _mean_lin_add(base, ssum, cnt2, w, b):
    def body(base_ref, s_ref, ca_ref, cb_ref, w_ref, b_ref, o_ref):
        c = jnp.maximum(ca_ref[...] + cb_ref[...], 1.0)
        m = s_ref[...] / jnp.concatenate([c, c], axis=1)
        o_ref[...] = base_ref[...] + jnp.maximum(
            jnp.dot(m, w_ref[...], preferred_element_type=jnp.float32)
            + b_ref[...], 0.0)
    r = base.shape[0]
    return pl.pallas_call(
        body, out_shape=jax.ShapeDtypeStruct((r, H), jnp.float32))(
            base, ssum, cnt2[0], cnt2[1], w, b.reshape(1, -1))


def _final(hs, hcnt2, cs, ccnt2, aw, ab, cw, cb, lw, lb):
    def body(hs_ref, hca_ref, hcb_ref, cs_ref, cca_ref, ccb_ref, aw_ref,
             ab_ref, cw_ref, cb_ref, lw_ref, lb_ref, o_ref):
        hc = jnp.maximum(hca_ref[...] + hcb_ref[...], 1.0)
        cc = jnp.maximum(cca_ref[...] + ccb_ref[...], 1.0)
        xg = jnp.dot(hs_ref[...] / jnp.concatenate([hc, hc], axis=1),
                     aw_ref[...], preferred_element_type=jnp.float32)
        xg = xg + ab_ref[...]
        xc = jnp.dot(cs_ref[...] / jnp.concatenate([cc, cc], axis=1),
                     cw_ref[...], preferred_element_type=jnp.float32)
        xc = xc + cb_ref[...]
        out = jnp.maximum(xg + xc, 0.0)
        o_ref[...] = jnp.dot(out, lw_ref[...],
                             preferred_element_type=jnp.float32) + lb_ref[...]
    g, out_dim = N_GRAPHS, lw.shape[1]
    return pl.pallas_call(
        body, out_shape=jax.ShapeDtypeStruct((g, out_dim), jnp.float32))(
            hs, hcnt2[0], hcnt2[1], cs, ccnt2[0], ccnt2[1], aw,
            ab.reshape(1, -1), cw, cb.reshape(1, -1), lw, lb.reshape(1, -1))


def kernel(params, clique_attr, x, edge_index, edge_attr, a2c_atom,
           a2c_clique, tree_edge_index, batch, tree_batch):
    p = params
    n = x.shape[0]
    ncl = clique_attr.shape[0]
    g = N_GRAPHS

    enc_g = (x.astype(jnp.int32)
             + 100 * jnp.arange(9, dtype=jnp.int32)[None, :]).reshape(-1)
    enc_s = jnp.repeat(jnp.arange(n, dtype=jnp.int32), 9)
    enc_gp, enc_sp = _pad_idx(enc_g, enc_s, n)

    src, dst = edge_index[0], edge_index[1]
    et = (edge_attr[:, 0] * 36 + edge_attr[:, 1] * 6
          + edge_attr[:, 2]).astype(jnp.int32)
    e_gp, e_sp, e_ep = _pad_idx(src, dst, n, et, unit=NS * 64 * 2)

    a2c_gp, a2c_sp = _pad_idx(a2c_atom, a2c_clique, ncl)
    c2a_gp, c2a_sp = _pad_idx(a2c_clique, a2c_atom, n)
    ts, td = tree_edge_index[0], tree_edge_index[1]
    t_gp, t_sp = _pad_idx(ts, td, ncl)
    ro_g = jnp.concatenate([jnp.arange(n, dtype=jnp.int32),
                            n + jnp.arange(ncl, dtype=jnp.int32)])
    ro_s = jnp.concatenate([batch.astype(jnp.int32),
                            g + tree_batch.astype(jnp.int32)])
    ro_gp, ro_sp = _pad_idx(ro_g, ro_s, 2 * g)

    off_at = ncl
    off_b = ncl + n
    off_tb = ncl + n + g
    n_out_c = ncl + n + 2 * g
    cnt_ids = jnp.concatenate([
        a2c_clique.astype(jnp.int32),
        off_at + a2c_atom.astype(jnp.int32),
        off_b + batch.astype(jnp.int32),
        off_tb + tree_batch.astype(jnp.int32)])
    ep = _round_up(cnt_ids.shape[0], NS * CH * 4)
    cnt_ids = jnp.concatenate([
        cnt_ids, jnp.full((ep - cnt_ids.shape[0],), n_out_c, jnp.int32)])
    n_acc_c = _round_up(n_out_c + 1, NS * 8)
    cnts = _sc_counts(cnt_ids, jnp.ones((CH, HALF), jnp.float32),
                      jnp.zeros((n_acc_c, HALF), jnp.float32), n_out=n_out_c)
    cnt_cl = (cnts[0, 0:ncl], cnts[1, 0:ncl])
    cnt_at = (cnts[0, off_at:off_at + n], cnts[1, off_at:off_at + n])
    cnt_b = (cnts[0, off_b:off_b + g], cnts[1, off_b:off_b + g])
    cnt_tb = (cnts[0, off_tb:off_tb + g], cnts[1, off_tb:off_tb + g])

    flat_emb = p['atom_emb'].reshape(9 * 100, H)
    h = _seg_sum(flat_emb, enc_gp, enc_sp, 900, n)
    x_cl = _clique_enc(clique_attr, p['clique_emb'])
    bond_tabs = _bond_tables(p['bond_emb'])

    for l in range(L):
        agg = _seg_sum(h, e_gp, e_sp, n, n, bond_tabs[l], e_ep, 216, ch=64)
        h = _mlp(h, agg, p['eps_atom'][l], p['atom_w1'][l], p['atom_b1'][l],
                 p['atom_g1'][l], p['atom_be1'][l], p['atom_w2'][l],
                 p['atom_b2'][l], p['atom_bn_g'][l], p['atom_bn_b'][l])
        s_m = _seg_sum(h, a2c_gp, a2c_sp, n, ncl)
        x_cl = _mean_lin_add(x_cl, s_m, cnt_cl, p['a2c_w'][l], p['a2c_b'][l])
        aggc = _seg_sum(x_cl, t_gp, t_sp, ncl, ncl)
        x_cl = _mlp(x_cl, aggc, p['eps_clique'][l], p['cl_w1'][l],
                    p['cl_b1'][l], p['cl_g1'][l], p['cl_be1'][l],
                    p['cl_w2'][l], p['cl_b2'][l], p['cl_bn_g'][l],
                    p['cl_bn_b'][l])
        s_mc = _seg_sum(x_cl, c2a_gp, c2a_sp, ncl, n)
        h = _mean_lin_add(h, s_mc, cnt_at, p['c2a_w'][l], p['c2a_b'][l])

    ro = _seg_sum(jnp.concatenate([h, x_cl], axis=0), ro_gp, ro_sp,
                  n + ncl, 2 * g)
    hs = ro[:g]
    cs = ro[g:]
    return _final(hs, cnt_b, cs, cnt_tb, p['atom_lin_w'], p['atom_lin_b'],
                  p['clique_lin_w'], p['clique_lin_b'], p['lin_w'], p['lin_b'])
